# baseline (speedup 1.0000x reference)
import jax
import jax.numpy as jnp
from jax.experimental import pallas as pl
from jax.experimental.pallas import tpu as pltpu

SEQ_LEN = 8192
D = 768
B = 4
EPS = 1e-12

BLOCK_ROWS = 2048


def _fused_ln_kernel(x_ref, p_ref, g_ref, b_ref, o_ref):
    x = x_ref[...] + p_ref[...]
    mean = jnp.mean(x, axis=-1, keepdims=True)
    xc = x - mean
    var = jnp.mean(xc * xc, axis=-1, keepdims=True)
    o_ref[...] = xc * jax.lax.rsqrt(var + EPS) * g_ref[...] + b_ref[...]


def _tc_part(x, pos_table, g, bt):
    b, s, d = x.shape
    grid = (s // BLOCK_ROWS, b)
    return pl.pallas_call(
        _fused_ln_kernel,
        grid=grid,
        in_specs=[
            pl.BlockSpec((1, BLOCK_ROWS, d), lambda i, j: (j, i, 0)),
            pl.BlockSpec((BLOCK_ROWS, d), lambda i, j: (i, 0)),
            pl.BlockSpec((1, d), lambda i, j: (0, 0)),
            pl.BlockSpec((1, d), lambda i, j: (0, 0)),
        ],
        out_specs=pl.BlockSpec((1, BLOCK_ROWS, d), lambda i, j: (j, i, 0)),
        out_shape=jax.ShapeDtypeStruct((b, s, d), x.dtype),
        compiler_params=pltpu.CompilerParams(
            dimension_semantics=("parallel", "parallel"),
        ),
    )(x, pos_table, g, bt)


def kernel(inputs_embeds, pos_table, ln_gamma, ln_beta):
    b, s, d = inputs_embeds.shape
    g = ln_gamma.reshape(1, d)
    bt = ln_beta.reshape(1, d)
    out0 = _tc_part(inputs_embeds[:3], pos_table, g, bt)
    out1 = _tc_part(inputs_embeds[3:], pos_table, g, bt)
    return jnp.concatenate([out0, out1], axis=0)


# speedup vs baseline: 2.7003x; 2.7003x over previous
import jax
import jax.numpy as jnp
from jax.experimental import pallas as pl
from jax.experimental.pallas import tpu as pltpu

SEQ_LEN = 8192
D = 768
B = 4
EPS = 1e-12

BLOCK_ROWS = 2048


def _fused_ln_kernel(x_ref, p_ref, g_ref, b_ref, o_ref):
    x = x_ref[...] + p_ref[...]
    mean = jnp.mean(x, axis=-1, keepdims=True)
    xc = x - mean
    var = jnp.mean(xc * xc, axis=-1, keepdims=True)
    o_ref[...] = xc * jax.lax.rsqrt(var + EPS) * g_ref[...] + b_ref[...]


def _tc_part(x, pos_table, g, bt):
    b, s, d = x.shape
    grid = (s // BLOCK_ROWS, b)
    return pl.pallas_call(
        _fused_ln_kernel,
        grid=grid,
        in_specs=[
            pl.BlockSpec((1, BLOCK_ROWS, d), lambda i, j: (j, i, 0)),
            pl.BlockSpec((BLOCK_ROWS, d), lambda i, j: (i, 0)),
            pl.BlockSpec((1, d), lambda i, j: (0, 0)),
            pl.BlockSpec((1, d), lambda i, j: (0, 0)),
        ],
        out_specs=pl.BlockSpec((1, BLOCK_ROWS, d), lambda i, j: (j, i, 0)),
        out_shape=jax.ShapeDtypeStruct((b, s, d), x.dtype),
        compiler_params=pltpu.CompilerParams(
            dimension_semantics=("parallel", "parallel"),
        ),
    )(x, pos_table, g, bt)


def kernel(inputs_embeds, pos_table, ln_gamma, ln_beta):
    g = ln_gamma.reshape(1, D)
    bt = ln_beta.reshape(1, D)
    return _tc_part(inputs_embeds, pos_table, g, bt)
